# baseline (speedup 1.0000x reference)
import functools

import jax
import jax.numpy as jnp
from jax import lax
from jax.experimental import pallas as pl
from jax.experimental.pallas import tpu as pltpu
from jax.experimental.pallas import tpu_sc as plsc

_N = 10000
_M = 16
_B = 4
_S = 132
_P = 9
_AF = 106
_NBR = 41
_FP = 128
_GW = 256
_TA = 1000
_TC = 400
_TE = _TC * _M
_TR = 1056
_FG = 4


def _sc_gather_rows(table, idx, chunk, nchunks):
    K = idx.shape[0]
    W = table.shape[1]
    dt = table.dtype
    per = chunk * nchunks
    assert per * 32 == K and chunk % 8 == 0
    mesh = plsc.VectorSubcoreMesh(core_axis_name="c", subcore_axis_name="s")

    NB = 3

    @functools.partial(
        pl.kernel,
        mesh=mesh,
        out_type=jax.ShapeDtypeStruct((K, W), dt),
        scratch_types=(
            [pltpu.VMEM((chunk,), jnp.int32) for _ in range(NB)]
            + [pltpu.VMEM((chunk, W), dt) for _ in range(NB)]
            + [pltpu.SemaphoreType.DMA for _ in range(2 * NB)]
        ),
    )
    def k(table_hbm, idx_hbm, out_hbm, *scr):
        ibufs = scr[0:NB]
        bufs = scr[NB:2 * NB]
        gsems = scr[2 * NB:3 * NB]
        wsems = scr[3 * NB:4 * NB]
        wid = lax.axis_index("s") * 2 + lax.axis_index("c")
        base = wid * per

        def start_gather(c):
            s = c % NB
            pltpu.sync_copy(idx_hbm.at[pl.ds(base + c * chunk, chunk)],
                            ibufs[s])
            pltpu.async_copy(table_hbm.at[ibufs[s]], bufs[s], gsems[s])

        for c in range(min(2, nchunks)):
            start_gather(c)
        for c in range(nchunks):
            s = c % NB
            pltpu.make_async_copy(
                table_hbm.at[ibufs[s]], bufs[s], gsems[s]
            ).wait()
            if c + 2 < nchunks:
                s2 = (c + 2) % NB
                if c >= 1:
                    pltpu.make_async_copy(
                        bufs[s2],
                        out_hbm.at[pl.ds(base + (c - 1) * chunk, chunk)],
                        wsems[s2],
                    ).wait()
                start_gather(c + 2)
            pltpu.async_copy(
                bufs[s], out_hbm.at[pl.ds(base + c * chunk, chunk)],
                wsems[s],
            )
        for c in range(max(0, nchunks - 3), nchunks):
            s = c % NB
            pltpu.make_async_copy(
                bufs[s], out_hbm.at[pl.ds(base + c * chunk, chunk)],
                wsems[s],
            ).wait()

    return k(table, idx)


def _embed_k(a_ref, w_ref, b_ref, o_ref):
    o_ref[...] = (
        lax.dot_general(
            a_ref[...], w_ref[...], (((0,), (0,)), ((), ())),
            preferred_element_type=jnp.float32,
        )
        + b_ref[...]
    )


def _embed(afea_t, Wp, bp):
    return pl.pallas_call(
        _embed_k,
        in_specs=[
            pl.BlockSpec((92, _N), lambda: (0, 0)),
            pl.BlockSpec((92, _FP), lambda: (0, 0)),
            pl.BlockSpec((1, _FP), lambda: (0, 0)),
        ],
        out_specs=pl.BlockSpec((_N, _FP), lambda: (0, 0)),
        out_shape=jax.ShapeDtypeStruct((_N, _FP), jnp.float32),
    )(afea_t, Wp, bp)


def _conv_edge_k(x_ref, xg_ref, nb_ref, wa_ref, wb0_ref, we_ref,
                 bf_ref, g_ref, st_ref):
    i = pl.program_id(0)
    u = (
        jnp.dot(x_ref[...], wa_ref[...], preferred_element_type=jnp.float32)
        + bf_ref[...]
    )
    urep = jnp.broadcast_to(u[:, None, :], (_TC, _M, _GW)).reshape(_TE, _GW)
    g = (
        urep
        + jnp.dot(xg_ref[...], wb0_ref[...], preferred_element_type=jnp.float32)
        + jnp.dot(nb_ref[...], we_ref[...], preferred_element_type=jnp.float32)
    )
    g_ref[...] = g.astype(jnp.bfloat16)
    ps = jnp.sum(g, axis=0, keepdims=True)
    pq = jnp.sum(g * g, axis=0, keepdims=True)
    st = jnp.concatenate([ps, pq], axis=0)

    @pl.when(i == 0)
    def _():
        st_ref[...] = st

    @pl.when(i != 0)
    def _():
        st_ref[...] = st_ref[...] + st


def _conv_edge(x, xg, nbr2, Wa, Wb0, We, bf):
    grid = _N // _TC
    return pl.pallas_call(
        _conv_edge_k,
        grid=(grid,),
        in_specs=[
            pl.BlockSpec((_TC, _FP), lambda i: (i, 0)),
            pl.BlockSpec((_TE, _FP), lambda i: (i, 0)),
            pl.BlockSpec((_TE, _NBR), lambda i: (i, 0)),
            pl.BlockSpec((_FP, _GW), lambda i: (0, 0)),
            pl.BlockSpec((_FP, _GW), lambda i: (0, 0)),
            pl.BlockSpec((_NBR, _GW), lambda i: (0, 0)),
            pl.BlockSpec((1, _GW), lambda i: (0, 0)),
        ],
        out_specs=[
            pl.BlockSpec((_TE, _GW), lambda i: (i, 0)),
            pl.BlockSpec((2, _GW), lambda i: (0, 0)),
        ],
        out_shape=[
            jax.ShapeDtypeStruct((_N * _M, _GW), jnp.bfloat16),
            jax.ShapeDtypeStruct((2, _GW), jnp.float32),
        ],
    )(x, xg, nbr2, Wa, Wb0, We, bf)


def _conv_bn_k(g_ref, st_ref, g1_ref, b1_ref, ns_ref, st2_ref):
    i = pl.program_id(0)
    st = st_ref[...]
    inv = 1.0 / (_N * _M)
    mean = st[0:1] * inv
    var = jnp.maximum(st[1:2] * inv - mean * mean, 0.0)
    scale = lax.rsqrt(var + 1e-5) * g1_ref[...]
    gbn = (g_ref[...].astype(jnp.float32) - mean) * scale + b1_ref[...]
    f = jax.nn.sigmoid(gbn[:, :_FP])
    c = jax.nn.softplus(gbn[:, _FP:])
    prod = f * c
    ns = jnp.sum(prod.reshape(_TC, _M, _FP), axis=1)
    ns_ref[...] = ns
    ps = jnp.sum(ns, axis=0, keepdims=True)
    pq = jnp.sum(ns * ns, axis=0, keepdims=True)
    st2 = jnp.concatenate([ps, pq], axis=0)

    @pl.when(i == 0)
    def _():
        st2_ref[...] = st2

    @pl.when(i != 0)
    def _():
        st2_ref[...] = st2_ref[...] + st2


def _conv_bn(gated, st, g1, b1):
    grid = _N // _TC
    return pl.pallas_call(
        _conv_bn_k,
        grid=(grid,),
        in_specs=[
            pl.BlockSpec((_TE, _GW), lambda i: (i, 0)),
            pl.BlockSpec((2, _GW), lambda i: (0, 0)),
            pl.BlockSpec((1, _GW), lambda i: (0, 0)),
            pl.BlockSpec((1, _GW), lambda i: (0, 0)),
        ],
        out_specs=[
            pl.BlockSpec((_TC, _FP), lambda i: (i, 0)),
            pl.BlockSpec((2, _FP), lambda i: (0, 0)),
        ],
        out_shape=[
            jax.ShapeDtypeStruct((_N, _FP), jnp.float32),
            jax.ShapeDtypeStruct((2, _FP), jnp.float32),
        ],
    )(gated, st, g1, b1)


def _conv_res_k(x_ref, ns_ref, st2_ref, g2_ref, b2_ref, o_ref):
    st2 = st2_ref[...]
    inv = 1.0 / _N
    m = st2[0:1] * inv
    v = jnp.maximum(st2[1:2] * inv - m * m, 0.0)
    nsn = (ns_ref[...] - m) * (lax.rsqrt(v + 1e-5) * g2_ref[...]) + b2_ref[...]
    o_ref[...] = jax.nn.softplus(x_ref[...] + nsn)


def _conv_res(x, ns, st2, g2, b2):
    return pl.pallas_call(
        _conv_res_k,
        grid=(_N // _TA,),
        in_specs=[
            pl.BlockSpec((_TA, _FP), lambda i: (i, 0)),
            pl.BlockSpec((_TA, _FP), lambda i: (i, 0)),
            pl.BlockSpec((2, _FP), lambda i: (0, 0)),
            pl.BlockSpec((1, _FP), lambda i: (0, 0)),
            pl.BlockSpec((1, _FP), lambda i: (0, 0)),
        ],
        out_specs=pl.BlockSpec((_TA, _FP), lambda i: (i, 0)),
        out_shape=jax.ShapeDtypeStruct((_N, _FP), jnp.float32),
    )(x, ns, st2, g2, b2)


def _mlp_k(li1_ref, li2_ref, tf_ref, hd_ref, w1a_ref, w1b_ref, b1_ref,
           w2_ref, b2_ref, pc_ref, o3_ref, cf_ref):
    lanes = lax.broadcasted_iota(jnp.int32, (_TR, 1024), 1)
    w = (
        (lanes == li1_ref[...]).astype(jnp.bfloat16)
        + (lanes == li2_ref[...]).astype(jnp.bfloat16)
    ) * jnp.bfloat16(0.5)
    avg = jnp.dot(w, tf_ref[...], preferred_element_type=jnp.float32)
    t1 = jax.nn.softplus(avg)
    t2 = jax.nn.softplus(hd_ref[...])
    h = jax.nn.softplus(
        jnp.dot(t1, w1a_ref[...], preferred_element_type=jnp.float32)
        + jnp.dot(t2, w1b_ref[...], preferred_element_type=jnp.float32)
        + b1_ref[...]
    )
    o = jnp.dot(h, w2_ref[...], preferred_element_type=jnp.float32) + b2_ref[...]
    o3_ref[...] = o
    sp2 = jax.nn.softplus(jnp.broadcast_to(o[:, 2:3], (_TR, 8)))
    c012 = jnp.exp(jnp.log(sp2) * pc_ref[...])
    o0b = jnp.broadcast_to(o[:, 0:1], (_TR, 8))
    o1b = jnp.broadcast_to(o[:, 1:2], (_TR, 8))
    li = lax.broadcasted_iota(jnp.int32, (_TR, 8), 1)
    cf_ref[...] = jnp.where(li < 3, c012, jnp.where(li == 3, o0b, o1b))


def _mlp(li1, li2, tf, hd2, W1a, W1b, b1, W2, b2, pc):
    nrow = _B * _S * _S
    grid = nrow // _TR
    return pl.pallas_call(
        _mlp_k,
        grid=(grid,),
        in_specs=[
            pl.BlockSpec((_TR, 1), lambda i: (i, 0)),
            pl.BlockSpec((_TR, 1), lambda i: (i, 0)),
            pl.BlockSpec((1024, _FP), lambda i: (0, 0)),
            pl.BlockSpec((_TR, _NBR), lambda i: (i, 0)),
            pl.BlockSpec((_FP, _FP), lambda i: (0, 0)),
            pl.BlockSpec((_NBR, _FP), lambda i: (0, 0)),
            pl.BlockSpec((1, _FP), lambda i: (0, 0)),
            pl.BlockSpec((_FP, 8), lambda i: (0, 0)),
            pl.BlockSpec((1, 8), lambda i: (0, 0)),
            pl.BlockSpec((1, 8), lambda i: (0, 0)),
        ],
        out_specs=[
            pl.BlockSpec((_TR, 8), lambda i: (i, 0)),
            pl.BlockSpec((_TR, 8), lambda i: (i, 0)),
        ],
        out_shape=[
            jax.ShapeDtypeStruct((nrow, 8), jnp.float32),
            jax.ShapeDtypeStruct((nrow, 8), jnp.float32),
        ],
    )(li1, li2, tf, hd2, W1a, W1b, b1, W2, b2, pc)


def _final_k(cr_ref, hop_ref, pss_ref, pds_ref, pdd_ref, pgds_ref, pgdd_ref,
             o_ref):
    cr = cr_ref[...]
    def e(kk):
        return cr[kk][:, None, None, :, :]
    acc = e(0) * pss_ref[...]
    acc = acc + e(1) * pds_ref[...]
    acc = acc + e(2) * pdd_ref[...]
    acc = acc + e(3) * pgds_ref[...]
    acc = acc + e(4) * pgdd_ref[...]
    o_ref[...] = hop_ref[...] * acc


def _final(crep, hop, pss, pds, pdd, pgds, pgdd):
    grid = _S // _FG
    t = pl.BlockSpec((_FG, _P, _P, _B, _S), lambda i: (i, 0, 0, 0, 0))
    return pl.pallas_call(
        _final_k,
        grid=(grid,),
        in_specs=[pl.BlockSpec((8, _FG, _B, _S), lambda i: (0, i, 0, 0)),
                  t, t, t, t, t, t],
        out_specs=t,
        out_shape=jax.ShapeDtypeStruct((_S, _P, _P, _B, _S), jnp.float32),
    )(crep, hop, pss, pds, pdd, pgds, pgdd)


def _pad_w(w, rows):
    out = jnp.zeros((rows, _GW), jnp.float32)
    out = out.at[: w.shape[0], :_AF].set(w[:, :_AF])
    out = out.at[: w.shape[0], _FP:_FP + _AF].set(w[:, _AF:])
    return out


def _pad_v212(v):
    out = jnp.zeros((1, _GW), jnp.float32)
    out = out.at[0, :_AF].set(v[:_AF])
    out = out.at[0, _FP:_FP + _AF].set(v[_AF:])
    return out


def _pad_v106(v):
    return jnp.zeros((1, _FP), jnp.float32).at[0, :_AF].set(v)


def kernel(atom_fea, nbr_fea, nbr_fea_idx, padding_filter, crystal_atom_idx,
           site_idx, batch_cif_ids, tabulated_hopping_distance,
           tabulated_hopping, tabulated_power_ss, tabulated_power_ds,
           tabulated_power_dd, tabulated_power_gamma_ds,
           tabulated_power_gamma_dd, params):
    f32 = jnp.float32
    afea_t = jnp.transpose(atom_fea.astype(f32))
    Wemb = jnp.zeros((92, _FP), f32).at[:, :_AF].set(params['W_emb'])
    bemb = _pad_v106(params['b_emb'])
    x = _embed(afea_t, Wemb, bemb)

    idxe = nbr_fea_idx.reshape(-1).astype(jnp.int32)
    idxe = jnp.concatenate([idxe, jnp.zeros((163840 - _N * _M,), jnp.int32)])
    nbr2 = nbr_fea.reshape(_N * _M, _NBR).astype(f32)

    for l in range(3):
        Wf = params['Wf%d' % l].astype(f32)
        Wa = _pad_w(Wf[0:_AF], _FP)
        Wb = _pad_w(Wf[_AF:2 * _AF], _FP)
        We = _pad_w(Wf[2 * _AF:], _NBR)
        bf = _pad_v212(params['bf%d' % l])
        g1 = _pad_v212(params['g1_%d' % l])
        b1 = _pad_v212(params['b1_%d' % l])
        g2 = _pad_v106(params['g2_%d' % l])
        b2 = _pad_v106(params['b2_%d' % l])
        xg = _sc_gather_rows(x, idxe, 320, 16)
        gated, st = _conv_edge(x, xg, nbr2, Wa, Wb, We, bf)
        ns, st2 = _conv_bn(gated, st, g1, b1)
        x = _conv_res(x, ns, st2, g2, b2)

    ci16 = crystal_atom_idx[:, :_M]
    t2 = nbr_fea_idx[ci16].astype(jnp.int32).reshape(-1)
    s0 = site_idx[..., 0].astype(jnp.int32)
    s1 = site_idx[..., 1].astype(jnp.int32)
    s2 = site_idx[..., 2].astype(jnp.int32)
    boff = (jnp.arange(_B, dtype=jnp.int32) * _M)[:, None, None]
    li1 = ((s0 + boff) * _M + s1).reshape(-1, 1)
    li2 = ((s0 + boff) * _M + s2).reshape(-1, 1)
    tf = _sc_gather_rows(x, t2, 32, 1)
    tf = tf.astype(jnp.bfloat16)

    hd2 = tabulated_hopping_distance.astype(f32).reshape(_B * _S * _S, _NBR)
    W1a = jnp.zeros((_FP, _FP), f32).at[:_AF, :].set(params['W1'][:_AF])
    W1b = params['W1'][_AF:].astype(f32)
    b1r = params['b1'].astype(f32).reshape(1, _FP)
    W2 = jnp.zeros((_FP, 8), f32).at[:, :3].set(params['W2'])
    b2r = jnp.zeros((1, 8), f32).at[0, :3].set(params['b2'])
    pc = jnp.zeros((1, 8), f32).at[0, :3].set(
        jnp.array([2.0 / 3.5, 1.0, 5.0 / 3.5], f32))
    o3p, cf = _mlp(li1, li2, tf, hd2, W1a, W1b, b1r, W2, b2r, pc)

    out3 = o3p[:, :3].reshape(_B, _S, _S, 3)
    crep = jnp.transpose(cf.reshape(_B, _S, _S, 8), (3, 1, 0, 2))

    v = lambda a: jnp.transpose(a.astype(f32), (1, 2, 4, 0, 3))
    fin = _final(crep, v(tabulated_hopping), v(tabulated_power_ss),
                 v(tabulated_power_ds), v(tabulated_power_dd),
                 v(tabulated_power_gamma_ds), v(tabulated_power_gamma_dd))
    return (jnp.transpose(fin, (3, 0, 1, 4, 2)), out3)


# speedup vs baseline: 1.0049x; 1.0049x over previous
import functools

import jax
import jax.numpy as jnp
from jax import lax
from jax.experimental import pallas as pl
from jax.experimental.pallas import tpu as pltpu
from jax.experimental.pallas import tpu_sc as plsc

_N = 10000
_M = 16
_B = 4
_S = 132
_P = 9
_AF = 106
_NBR = 41
_FP = 128
_GW = 256
_TA = 1000
_TC = 400
_TE = _TC * _M
_TR = 1056
_FG = 4


def _sc_gather_rows(table, idx, chunk, nchunks):
    K = idx.shape[0]
    W = table.shape[1]
    dt = table.dtype
    per = chunk * nchunks
    assert per * 32 == K and chunk % 8 == 0
    mesh = plsc.VectorSubcoreMesh(core_axis_name="c", subcore_axis_name="s")

    NB = 3

    @functools.partial(
        pl.kernel,
        mesh=mesh,
        out_type=jax.ShapeDtypeStruct((K, W), dt),
        scratch_types=(
            [pltpu.VMEM((chunk,), jnp.int32) for _ in range(NB)]
            + [pltpu.VMEM((chunk, W), dt) for _ in range(NB)]
            + [pltpu.SemaphoreType.DMA for _ in range(2 * NB)]
        ),
    )
    def k(table_hbm, idx_hbm, out_hbm, *scr):
        ibufs = scr[0:NB]
        bufs = scr[NB:2 * NB]
        gsems = scr[2 * NB:3 * NB]
        wsems = scr[3 * NB:4 * NB]
        wid = lax.axis_index("s") * 2 + lax.axis_index("c")
        base = wid * per

        def start_gather(c):
            s = c % NB
            pltpu.sync_copy(idx_hbm.at[pl.ds(base + c * chunk, chunk)],
                            ibufs[s])
            pltpu.async_copy(table_hbm.at[ibufs[s]], bufs[s], gsems[s])

        for c in range(min(2, nchunks)):
            start_gather(c)
        for c in range(nchunks):
            s = c % NB
            pltpu.make_async_copy(
                table_hbm.at[ibufs[s]], bufs[s], gsems[s]
            ).wait()
            if c + 2 < nchunks:
                s2 = (c + 2) % NB
                if c >= 1:
                    pltpu.make_async_copy(
                        bufs[s2],
                        out_hbm.at[pl.ds(base + (c - 1) * chunk, chunk)],
                        wsems[s2],
                    ).wait()
                start_gather(c + 2)
            pltpu.async_copy(
                bufs[s], out_hbm.at[pl.ds(base + c * chunk, chunk)],
                wsems[s],
            )
        for c in range(max(0, nchunks - 3), nchunks):
            s = c % NB
            pltpu.make_async_copy(
                bufs[s], out_hbm.at[pl.ds(base + c * chunk, chunk)],
                wsems[s],
            ).wait()

    return k(table, idx)


def _embed_k(a_ref, w_ref, b_ref, o_ref):
    o_ref[...] = (
        lax.dot_general(
            a_ref[...], w_ref[...], (((0,), (0,)), ((), ())),
            preferred_element_type=jnp.float32,
        )
        + b_ref[...]
    )


def _embed(afea_t, Wp, bp):
    return pl.pallas_call(
        _embed_k,
        in_specs=[
            pl.BlockSpec((92, _N), lambda: (0, 0)),
            pl.BlockSpec((92, _FP), lambda: (0, 0)),
            pl.BlockSpec((1, _FP), lambda: (0, 0)),
        ],
        out_specs=pl.BlockSpec((_N, _FP), lambda: (0, 0)),
        out_shape=jax.ShapeDtypeStruct((_N, _FP), jnp.float32),
    )(afea_t, Wp, bp)


def _conv_edge_k(x_ref, xg_ref, nb_ref, wa_ref, wb0_ref, we_ref,
                 bf_ref, g_ref, st_ref):
    i = pl.program_id(0)
    u = (
        jnp.dot(x_ref[...], wa_ref[...], preferred_element_type=jnp.float32)
        + bf_ref[...]
    )
    urep = jnp.broadcast_to(u[:, None, :], (_TC, _M, _GW)).reshape(_TE, _GW)
    g = (
        urep
        + jnp.dot(xg_ref[...], wb0_ref[...], preferred_element_type=jnp.float32)
        + jnp.dot(nb_ref[...], we_ref[...], preferred_element_type=jnp.float32)
    )
    g_ref[...] = g.astype(jnp.bfloat16)
    ps = jnp.sum(g, axis=0, keepdims=True)
    pq = jnp.sum(g * g, axis=0, keepdims=True)
    st = jnp.concatenate([ps, pq], axis=0)

    @pl.when(i == 0)
    def _():
        st_ref[...] = st

    @pl.when(i != 0)
    def _():
        st_ref[...] = st_ref[...] + st


def _conv_edge(x, xg, nbr2, Wa, Wb0, We, bf):
    grid = _N // _TC
    return pl.pallas_call(
        _conv_edge_k,
        grid=(grid,),
        in_specs=[
            pl.BlockSpec((_TC, _FP), lambda i: (i, 0)),
            pl.BlockSpec((_TE, _FP), lambda i: (i, 0)),
            pl.BlockSpec((_TE, _NBR), lambda i: (i, 0)),
            pl.BlockSpec((_FP, _GW), lambda i: (0, 0)),
            pl.BlockSpec((_FP, _GW), lambda i: (0, 0)),
            pl.BlockSpec((_NBR, _GW), lambda i: (0, 0)),
            pl.BlockSpec((1, _GW), lambda i: (0, 0)),
        ],
        out_specs=[
            pl.BlockSpec((_TE, _GW), lambda i: (i, 0)),
            pl.BlockSpec((2, _GW), lambda i: (0, 0)),
        ],
        out_shape=[
            jax.ShapeDtypeStruct((_N * _M, _GW), jnp.bfloat16),
            jax.ShapeDtypeStruct((2, _GW), jnp.float32),
        ],
    )(x, xg, nbr2, Wa, Wb0, We, bf)


def _conv_bn_k(g_ref, st_ref, g1_ref, b1_ref, ns_ref, st2_ref):
    i = pl.program_id(0)
    st = st_ref[...]
    inv = 1.0 / (_N * _M)
    mean = st[0:1] * inv
    var = jnp.maximum(st[1:2] * inv - mean * mean, 0.0)
    scale = lax.rsqrt(var + 1e-5) * g1_ref[...]
    gbn = (g_ref[...].astype(jnp.float32) - mean) * scale + b1_ref[...]
    f = jax.nn.sigmoid(gbn[:, :_FP])
    c = jax.nn.softplus(gbn[:, _FP:])
    prod = f * c
    ns = jnp.sum(prod.reshape(_TC, _M, _FP), axis=1)
    ns_ref[...] = ns
    ps = jnp.sum(ns, axis=0, keepdims=True)
    pq = jnp.sum(ns * ns, axis=0, keepdims=True)
    st2 = jnp.concatenate([ps, pq], axis=0)

    @pl.when(i == 0)
    def _():
        st2_ref[...] = st2

    @pl.when(i != 0)
    def _():
        st2_ref[...] = st2_ref[...] + st2


def _conv_bn(gated, st, g1, b1):
    grid = _N // _TC
    return pl.pallas_call(
        _conv_bn_k,
        grid=(grid,),
        in_specs=[
            pl.BlockSpec((_TE, _GW), lambda i: (i, 0)),
            pl.BlockSpec((2, _GW), lambda i: (0, 0)),
            pl.BlockSpec((1, _GW), lambda i: (0, 0)),
            pl.BlockSpec((1, _GW), lambda i: (0, 0)),
        ],
        out_specs=[
            pl.BlockSpec((_TC, _FP), lambda i: (i, 0)),
            pl.BlockSpec((2, _FP), lambda i: (0, 0)),
        ],
        out_shape=[
            jax.ShapeDtypeStruct((_N, _FP), jnp.float32),
            jax.ShapeDtypeStruct((2, _FP), jnp.float32),
        ],
    )(gated, st, g1, b1)


def _conv_res_k(x_ref, ns_ref, st2_ref, g2_ref, b2_ref, o_ref):
    st2 = st2_ref[...]
    inv = 1.0 / _N
    m = st2[0:1] * inv
    v = jnp.maximum(st2[1:2] * inv - m * m, 0.0)
    nsn = (ns_ref[...] - m) * (lax.rsqrt(v + 1e-5) * g2_ref[...]) + b2_ref[...]
    o_ref[...] = jax.nn.softplus(x_ref[...] + nsn)


def _conv_res(x, ns, st2, g2, b2):
    return pl.pallas_call(
        _conv_res_k,
        grid=(_N // _TA,),
        in_specs=[
            pl.BlockSpec((_TA, _FP), lambda i: (i, 0)),
            pl.BlockSpec((_TA, _FP), lambda i: (i, 0)),
            pl.BlockSpec((2, _FP), lambda i: (0, 0)),
            pl.BlockSpec((1, _FP), lambda i: (0, 0)),
            pl.BlockSpec((1, _FP), lambda i: (0, 0)),
        ],
        out_specs=pl.BlockSpec((_TA, _FP), lambda i: (i, 0)),
        out_shape=jax.ShapeDtypeStruct((_N, _FP), jnp.float32),
    )(x, ns, st2, g2, b2)


def _mlp_k(li1_ref, li2_ref, tf_ref, hd_ref, w1a_ref, w1b_ref, b1_ref,
           w2_ref, b2_ref, pc_ref, o3_ref, cf_ref):
    lanes = lax.broadcasted_iota(jnp.int32, (_TR, 1024), 1)
    w = (
        (lanes == li1_ref[...]).astype(jnp.float32)
        + (lanes == li2_ref[...]).astype(jnp.float32)
    ) * 0.5
    avg = jnp.dot(w, tf_ref[...], preferred_element_type=jnp.float32)
    t1 = jax.nn.softplus(avg)
    t2 = jax.nn.softplus(hd_ref[...])
    h = jax.nn.softplus(
        jnp.dot(t1, w1a_ref[...], preferred_element_type=jnp.float32)
        + jnp.dot(t2, w1b_ref[...], preferred_element_type=jnp.float32)
        + b1_ref[...]
    )
    o = jnp.dot(h, w2_ref[...], preferred_element_type=jnp.float32) + b2_ref[...]
    o3_ref[...] = o
    sp2 = jax.nn.softplus(jnp.broadcast_to(o[:, 2:3], (_TR, 8)))
    c012 = jnp.exp(jnp.log(sp2) * pc_ref[...])
    o0b = jnp.broadcast_to(o[:, 0:1], (_TR, 8))
    o1b = jnp.broadcast_to(o[:, 1:2], (_TR, 8))
    li = lax.broadcasted_iota(jnp.int32, (_TR, 8), 1)
    cf_ref[...] = jnp.where(li < 3, c012, jnp.where(li == 3, o0b, o1b))


def _mlp(li1, li2, tf, hd2, W1a, W1b, b1, W2, b2, pc):
    nrow = _B * _S * _S
    grid = nrow // _TR
    return pl.pallas_call(
        _mlp_k,
        grid=(grid,),
        in_specs=[
            pl.BlockSpec((_TR, 1), lambda i: (i, 0)),
            pl.BlockSpec((_TR, 1), lambda i: (i, 0)),
            pl.BlockSpec((1024, _FP), lambda i: (0, 0)),
            pl.BlockSpec((_TR, _NBR), lambda i: (i, 0)),
            pl.BlockSpec((_FP, _FP), lambda i: (0, 0)),
            pl.BlockSpec((_NBR, _FP), lambda i: (0, 0)),
            pl.BlockSpec((1, _FP), lambda i: (0, 0)),
            pl.BlockSpec((_FP, 8), lambda i: (0, 0)),
            pl.BlockSpec((1, 8), lambda i: (0, 0)),
            pl.BlockSpec((1, 8), lambda i: (0, 0)),
        ],
        out_specs=[
            pl.BlockSpec((_TR, 8), lambda i: (i, 0)),
            pl.BlockSpec((_TR, 8), lambda i: (i, 0)),
        ],
        out_shape=[
            jax.ShapeDtypeStruct((nrow, 8), jnp.float32),
            jax.ShapeDtypeStruct((nrow, 8), jnp.float32),
        ],
    )(li1, li2, tf, hd2, W1a, W1b, b1, W2, b2, pc)


def _final_k(cr_ref, hop_ref, pss_ref, pds_ref, pdd_ref, pgds_ref, pgdd_ref,
             o_ref):
    cr = cr_ref[...]
    def e(kk):
        return cr[kk][:, None, None, :, :]
    acc = e(0) * pss_ref[...]
    acc = acc + e(1) * pds_ref[...]
    acc = acc + e(2) * pdd_ref[...]
    acc = acc + e(3) * pgds_ref[...]
    acc = acc + e(4) * pgdd_ref[...]
    o_ref[...] = hop_ref[...] * acc


def _final(crep, hop, pss, pds, pdd, pgds, pgdd):
    grid = _S // _FG
    t = pl.BlockSpec((_FG, _P, _P, _B, _S), lambda i: (i, 0, 0, 0, 0))
    return pl.pallas_call(
        _final_k,
        grid=(grid,),
        in_specs=[pl.BlockSpec((8, _FG, _B, _S), lambda i: (0, i, 0, 0)),
                  t, t, t, t, t, t],
        out_specs=t,
        out_shape=jax.ShapeDtypeStruct((_S, _P, _P, _B, _S), jnp.float32),
    )(crep, hop, pss, pds, pdd, pgds, pgdd)


def _pad_w(w, rows):
    out = jnp.zeros((rows, _GW), jnp.float32)
    out = out.at[: w.shape[0], :_AF].set(w[:, :_AF])
    out = out.at[: w.shape[0], _FP:_FP + _AF].set(w[:, _AF:])
    return out


def _pad_v212(v):
    out = jnp.zeros((1, _GW), jnp.float32)
    out = out.at[0, :_AF].set(v[:_AF])
    out = out.at[0, _FP:_FP + _AF].set(v[_AF:])
    return out


def _pad_v106(v):
    return jnp.zeros((1, _FP), jnp.float32).at[0, :_AF].set(v)


def kernel(atom_fea, nbr_fea, nbr_fea_idx, padding_filter, crystal_atom_idx,
           site_idx, batch_cif_ids, tabulated_hopping_distance,
           tabulated_hopping, tabulated_power_ss, tabulated_power_ds,
           tabulated_power_dd, tabulated_power_gamma_ds,
           tabulated_power_gamma_dd, params):
    f32 = jnp.float32
    afea_t = jnp.transpose(atom_fea.astype(f32))
    Wemb = jnp.zeros((92, _FP), f32).at[:, :_AF].set(params['W_emb'])
    bemb = _pad_v106(params['b_emb'])
    x = _embed(afea_t, Wemb, bemb)

    idxe = nbr_fea_idx.reshape(-1).astype(jnp.int32)
    idxe = jnp.concatenate([idxe, jnp.zeros((163840 - _N * _M,), jnp.int32)])
    nbr2 = nbr_fea.reshape(_N * _M, _NBR).astype(f32)

    for l in range(3):
        Wf = params['Wf%d' % l].astype(f32)
        Wa = _pad_w(Wf[0:_AF], _FP)
        Wb = _pad_w(Wf[_AF:2 * _AF], _FP)
        We = _pad_w(Wf[2 * _AF:], _NBR)
        bf = _pad_v212(params['bf%d' % l])
        g1 = _pad_v212(params['g1_%d' % l])
        b1 = _pad_v212(params['b1_%d' % l])
        g2 = _pad_v106(params['g2_%d' % l])
        b2 = _pad_v106(params['b2_%d' % l])
        xg = _sc_gather_rows(x, idxe, 320, 16)
        gated, st = _conv_edge(x, xg, nbr2, Wa, Wb, We, bf)
        ns, st2 = _conv_bn(gated, st, g1, b1)
        x = _conv_res(x, ns, st2, g2, b2)

    ci16 = crystal_atom_idx[:, :_M]
    t2 = nbr_fea_idx[ci16].astype(jnp.int32).reshape(-1)
    s0 = site_idx[..., 0].astype(jnp.int32)
    s1 = site_idx[..., 1].astype(jnp.int32)
    s2 = site_idx[..., 2].astype(jnp.int32)
    boff = (jnp.arange(_B, dtype=jnp.int32) * _M)[:, None, None]
    li1 = ((s0 + boff) * _M + s1).reshape(-1, 1)
    li2 = ((s0 + boff) * _M + s2).reshape(-1, 1)
    tf = _sc_gather_rows(x, t2, 32, 1)

    hd2 = tabulated_hopping_distance.astype(f32).reshape(_B * _S * _S, _NBR)
    W1a = jnp.zeros((_FP, _FP), f32).at[:_AF, :].set(params['W1'][:_AF])
    W1b = params['W1'][_AF:].astype(f32)
    b1r = params['b1'].astype(f32).reshape(1, _FP)
    W2 = jnp.zeros((_FP, 8), f32).at[:, :3].set(params['W2'])
    b2r = jnp.zeros((1, 8), f32).at[0, :3].set(params['b2'])
    pc = jnp.zeros((1, 8), f32).at[0, :3].set(
        jnp.array([2.0 / 3.5, 1.0, 5.0 / 3.5], f32))
    o3p, cf = _mlp(li1, li2, tf, hd2, W1a, W1b, b1r, W2, b2r, pc)

    out3 = o3p[:, :3].reshape(_B, _S, _S, 3)
    crep = jnp.transpose(cf.reshape(_B, _S, _S, 8), (3, 1, 0, 2))

    v = lambda a: jnp.transpose(a.astype(f32), (1, 2, 4, 0, 3))
    fin = _final(crep, v(tabulated_hopping), v(tabulated_power_ss),
                 v(tabulated_power_ds), v(tabulated_power_dd),
                 v(tabulated_power_gamma_ds), v(tabulated_power_gamma_dd))
    return (jnp.transpose(fin, (3, 0, 1, 4, 2)), out3)
